# Initial kernel scaffold; baseline (speedup 1.0000x reference)
import jax
import jax.numpy as jnp
from jax.experimental import pallas as pl


def kernel(memory, node_idxs, values):
    raise NotImplementedError("write your pallas kernel here")


# speedup vs baseline: 18.1578x; 18.1578x over previous
import jax
import jax.numpy as jnp
from jax import lax
from jax.experimental import pallas as pl
from jax.experimental.pallas import tpu as pltpu
from jax.experimental.pallas import tpu_sc as plsc

N_NODES = 1_000_000
MEM_DIM = 64
BATCH = 16384

NC = 2
NS = 16
L = 16
NW = NC * NS
LOGR = 15
RANGE = 1 << LOGR
CHUNK = 128
EBUF = BATCH + CHUNK


def _body(idx_hbm, val_hbm, out_hbm, idx_v, tab_v, jb_v, wb_v, rows_v, sem):
    c = lax.axis_index("c")
    s = lax.axis_index("s")
    wid = s * NC + c

    pltpu.sync_copy(idx_hbm, idx_v)

    iota = lax.iota(jnp.int32, L)

    def scan_tab(i, carry):
        v = idx_v[pl.ds(i * L, L)]
        pos = iota + i * L
        m = lax.shift_right_logical(v, LOGR) == wid
        loc = lax.bitwise_and(v, RANGE - 1)
        plsc.store_scatter(tab_v, [loc], pos, mask=m)

        def wbody(_):
            g = plsc.load_gather(tab_v, [loc], mask=m)
            need = jnp.logical_and(m, pos > g)
            plsc.store_scatter(tab_v, [loc], pos, mask=need)
            return jnp.max(plsc.all_reduce_population_count(need))

        lax.while_loop(lambda n: n > 0, wbody, jnp.int32(1))
        return carry

    lax.fori_loop(0, BATCH // L, scan_tab, 0)

    def scan_emit(i, cnt):
        v = idx_v[pl.ds(i * L, L)]
        pos = iota + i * L
        m = lax.shift_right_logical(v, LOGR) == wid
        loc = lax.bitwise_and(v, RANGE - 1)
        w = plsc.load_gather(tab_v, [loc], mask=m)
        plsc.store_compressed(jb_v.at[pl.ds(cnt, L)], pos, mask=m)
        plsc.store_compressed(wb_v.at[pl.ds(cnt, L)], w, mask=m)
        return cnt + jnp.max(plsc.all_reduce_population_count(m))

    cnt = lax.fori_loop(0, BATCH // L, scan_emit, jnp.int32(0))

    for q in range(CHUNK // L):
        pad = iota + q * L
        jb_v[pl.ds(cnt + q * L, L)] = pad + BATCH
        wb_v[pl.ds(cnt + q * L, L)] = pad + (wid * CHUNK)

    nch = lax.shift_right_logical(cnt + CHUNK - 1, 7)

    def chunk(k, carry):
        off = k * CHUNK
        pltpu.async_copy(
            val_hbm.at[wb_v.at[pl.ds(off, CHUNK)]], rows_v, sem
        ).wait()
        pltpu.async_copy(
            rows_v, out_hbm.at[jb_v.at[pl.ds(off, CHUNK)]], sem
        ).wait()
        return carry

    lax.fori_loop(0, nch, chunk, 0)


_sc_call = pl.kernel(
    _body,
    out_type=jax.ShapeDtypeStruct((BATCH + CHUNK, MEM_DIM), jnp.float32),
    mesh=plsc.VectorSubcoreMesh(
        core_axis_name="c", subcore_axis_name="s", num_cores=NC
    ),
    scratch_types=[
        pltpu.VMEM((BATCH,), jnp.int32),
        pltpu.VMEM((RANGE,), jnp.int32),
        pltpu.VMEM((EBUF,), jnp.int32),
        pltpu.VMEM((EBUF,), jnp.int32),
        pltpu.VMEM((CHUNK, MEM_DIM), jnp.float32),
        pltpu.SemaphoreType.DMA,
    ],
    compiler_params=pltpu.CompilerParams(
        needs_layout_passes=False, use_tc_tiling_on_sc=False
    ),
)


def kernel(memory, node_idxs, values):
    del memory
    out = _sc_call(node_idxs.astype(jnp.int32), values)
    return out[:BATCH]
